# baseline (speedup 1.0000x reference)
import functools

import jax
import jax.numpy as jnp
from jax import lax
from jax.experimental import pallas as pl
from jax.experimental.pallas import tpu as pltpu
from jax.experimental.pallas import tpu_sc as plsc

NC = 2
NS = 16
NW = NC * NS
CH = 250
DEGW = 8


ZCH = 128


@functools.lru_cache(maxsize=None)
def _make_prop(NT, NP, D, E, with_deg):
  DH = D // 2
  assert E % (NS * CH) == 0
  NCH = E // (NS * CH)
  assert NP % (NS * ZCH) == 0
  RPT = NP // NS
  ZREP = RPT // ZCH

  mesh = plsc.VectorSubcoreMesh(core_axis_name="c", subcore_axis_name="s")

  out_type = [jax.ShapeDtypeStruct((NC, NP, DH), jnp.float32)]
  if with_deg:
    out_type.append(jax.ShapeDtypeStruct((NC, NP, DEGW), jnp.float32))

  NBUF = 2
  assert NCH % NBUF == 0
  NITER = NCH // NBUF

  scratch = [
      pltpu.VMEM((NCH, CH), jnp.int32),
      pltpu.VMEM((NCH, CH), jnp.int32),
      pltpu.VMEM((NBUF, CH, DH), jnp.float32),
      pltpu.VMEM((ZCH, DH), jnp.float32),
      pltpu.VMEM((CH, DEGW), jnp.float32),
      pltpu.VMEM_SHARED((NP, DH), jnp.float32),
      pltpu.VMEM_SHARED((NP, DEGW), jnp.float32),
  ] + [pltpu.SemaphoreType.DMA] * (2 * NBUF + 1)

  def body(src_hbm, dst_hbm, g_hbm, *rest):
    if with_deg:
      ones_hbm, zdeg_hbm = rest[0], rest[1]
      out_hbm, deg_hbm = rest[2], rest[3]
      scr = rest[4:]
    else:
      ones_hbm = zdeg_hbm = deg_hbm = None
      out_hbm = rest[0]
      scr = rest[1:]
    (src_v, dst_v, rows_v, zrow_v, ones_v, acc_sh, deg_sh) = scr[:7]
    gsems = scr[7:7 + NBUF]
    ssems = scr[7 + NBUF:7 + 2 * NBUF]
    dsem = scr[7 + 2 * NBUF]

    c = lax.axis_index("c")
    s = lax.axis_index("s")

    z16 = jnp.zeros((16,), jnp.float32)

    def fill_z(i, carry):
      for j in range(DH // 16):
        zrow_v[i, pl.ds(j * 16, 16)] = z16
      return carry

    lax.fori_loop(0, ZCH, fill_z, 0)
    if with_deg:
      pltpu.sync_copy(ones_hbm, ones_v)

    row0 = s * RPT
    for k in range(ZREP):
      pltpu.sync_copy(zrow_v, acc_sh.at[pl.ds(row0 + k * ZCH, ZCH)])
      if with_deg:
        pltpu.sync_copy(zdeg_hbm, deg_sh.at[pl.ds(row0 + k * ZCH, ZCH)])
    plsc.subcore_barrier()

    pltpu.sync_copy(src_hbm.at[s], src_v)
    pltpu.sync_copy(dst_hbm.at[s], dst_v)

    def start_gather(k, b):
      pltpu.async_copy(g_hbm.at[c].at[src_v.at[k]], rows_v.at[b], gsems[b])

    for b in range(NBUF):
      start_gather(b, b)

    def pipe_body(j, carry):
      base = j * NBUF
      for b in range(NBUF):
        k = base + b
        pltpu.make_async_copy(g_hbm.at[c].at[src_v.at[k]], rows_v.at[b],
                              gsems[b]).wait()
        pltpu.async_copy(rows_v.at[b], acc_sh.at[dst_v.at[k]], ssems[b],
                         add=True)
        if with_deg:
          @pl.when((c == 0) == (k < NCH // 2))
          def _():
            pltpu.async_copy(ones_v, deg_sh.at[dst_v.at[k]], dsem,
                             add=True)
      for b in range(NBUF):
        k = base + b
        pltpu.make_async_copy(rows_v.at[b], acc_sh.at[dst_v.at[k]],
                              ssems[b]).wait()

        @pl.when(j < NITER - 1)
        def _():
          start_gather(k + NBUF, b)
      return carry

    lax.fori_loop(0, NITER, pipe_body, 0)

    if with_deg:
      def drain(k, carry):
        pltpu.make_async_copy(ones_v, deg_sh.at[dst_v.at[0]], dsem).wait()
        return carry
      lax.fori_loop(0, NCH // 2, drain, 0)

    plsc.subcore_barrier()

    pltpu.sync_copy(acc_sh.at[pl.ds(row0, RPT)],
                    out_hbm.at[c, pl.ds(row0, RPT), :])
    if with_deg:
      pltpu.sync_copy(deg_sh.at[pl.ds(row0, RPT)],
                      deg_hbm.at[c, pl.ds(row0, RPT), :])

  return pl.kernel(body, out_type=out_type, mesh=mesh, scratch_types=scratch,
                   compiler_params=pltpu.CompilerParams(
                       use_tc_tiling_on_sc=False),
                   name="sc_prop_deg" if with_deg else "sc_prop")


@functools.lru_cache(maxsize=None)
def _make_combine(N, D, OUT, BN, split_out):
  assert N % BN == 0

  DH = D // 2
  OH = OUT // 2

  def body(acc_ref, deg_ref, w_ref, b_ref, o_ref):
    deg = deg_ref[0, :, 0:1] + deg_ref[1, :, 0:1]
    scale = 1.0 / jnp.maximum(deg, 1.0)
    a0 = acc_ref[0] * scale
    a1 = acc_ref[1] * scale
    g = jnp.maximum(
        jnp.dot(a0, w_ref[:DH, :], preferred_element_type=jnp.float32)
        + jnp.dot(a1, w_ref[DH:, :], preferred_element_type=jnp.float32)
        + b_ref[...], 0.0)
    if split_out:
      o_ref[0] = g[:, :OH]
      o_ref[1] = g[:, OH:]
    else:
      o_ref[...] = g

  if split_out:
    out_spec = pl.BlockSpec((NC, BN, OH), lambda i: (0, i, 0))
    out_shape = jax.ShapeDtypeStruct((NC, N, OH), jnp.float32)
  else:
    out_spec = pl.BlockSpec((BN, OUT), lambda i: (i, 0))
    out_shape = jax.ShapeDtypeStruct((N, OUT), jnp.float32)

  return pl.pallas_call(
      body,
      grid=(N // BN,),
      in_specs=[
          pl.BlockSpec((NC, BN, DH), lambda i: (0, i, 0)),
          pl.BlockSpec((NC, BN, DEGW), lambda i: (0, i, 0)),
          pl.BlockSpec((D, OUT), lambda i: (0, 0)),
          pl.BlockSpec((1, OUT), lambda i: (0, 0)),
      ],
      out_specs=out_spec,
      out_shape=out_shape,
      name="tc_combine_split" if split_out else "tc_combine",
  )


@functools.lru_cache(maxsize=None)
def _make_gru(N, D, OUT, BN):
  assert N % BN == 0
  DH = D // 2

  def body(acc_ref, deg_ref, w_ref, b_ref, h_ref, wx_ref, bx_ref, wh_ref,
           bh_ref, o_ref):
    deg = deg_ref[0, :, 0:1] + deg_ref[1, :, 0:1]
    scale = 1.0 / jnp.maximum(deg, 1.0)
    a0 = acc_ref[0] * scale
    a1 = acc_ref[1] * scale
    g = jnp.maximum(
        jnp.dot(a0, w_ref[:DH, :], preferred_element_type=jnp.float32)
        + jnp.dot(a1, w_ref[DH:, :], preferred_element_type=jnp.float32)
        + b_ref[...], 0.0)
    h = h_ref[...]
    gate_x = jnp.dot(g, wx_ref[...],
                     preferred_element_type=jnp.float32) + bx_ref[...]
    gate_h = jnp.dot(h, wh_ref[...],
                     preferred_element_type=jnp.float32) + bh_ref[...]
    i_r = gate_x[:, :OUT]
    i_u = gate_x[:, OUT:2 * OUT]
    i_n = gate_x[:, 2 * OUT:]
    h_r = gate_h[:, :OUT]
    h_u = gate_h[:, OUT:2 * OUT]
    h_n = gate_h[:, 2 * OUT:]
    r = jax.nn.sigmoid(i_r + h_r)
    u = jax.nn.sigmoid(i_u + h_u)
    cand = jnp.tanh(i_n + r * h_n)
    o_ref[...] = cand + u * (h - cand)

  return pl.pallas_call(
      body,
      grid=(N // BN,),
      in_specs=[
          pl.BlockSpec((NC, BN, DH), lambda i: (0, i, 0)),
          pl.BlockSpec((NC, BN, DEGW), lambda i: (0, i, 0)),
          pl.BlockSpec((D, OUT), lambda i: (0, 0)),
          pl.BlockSpec((1, OUT), lambda i: (0, 0)),
          pl.BlockSpec((BN, OUT), lambda i: (i, 0)),
          pl.BlockSpec((OUT, 3 * OUT), lambda i: (0, 0)),
          pl.BlockSpec((1, 3 * OUT), lambda i: (0, 0)),
          pl.BlockSpec((OUT, 3 * OUT), lambda i: (0, 0)),
          pl.BlockSpec((1, 3 * OUT), lambda i: (0, 0)),
      ],
      out_specs=pl.BlockSpec((BN, OUT), lambda i: (i, 0)),
      out_shape=jax.ShapeDtypeStruct((N, OUT), jnp.float32),
      name="tc_gru",
  )


def kernel(x_list, adj_list, W_gcn, b_gcn, Wx, bx, Wh, bh):
  T, N, D = x_list.shape
  L, _, OUT = W_gcn.shape
  E = adj_list.shape[2]
  ALIGN = NS * ZCH
  NP = ((N + ALIGN - 1) // ALIGN) * ALIGN
  BN = 1024
  assert NP % BN == 0
  NCH = E // (NS * CH)
  DH = D // 2

  prop_deg = _make_prop(N, NP, D, E, True)
  prop = _make_prop(NP, NP, OUT, E, False)
  combine_split = _make_combine(NP, D, OUT, BN, True)
  gru = _make_gru(NP, D, OUT, BN)

  b_gcn2 = b_gcn.reshape(L, 1, OUT)
  bx2 = bx.reshape(1, 3 * OUT)
  bh2 = bh.reshape(1, 3 * OUT)

  ones_deg = jnp.ones((CH, DEGW), jnp.float32)
  zeros_deg = jnp.zeros((ZCH, DEGW), jnp.float32)
  h = jnp.zeros((NP, OUT), x_list.dtype)
  outs = []
  for t in range(T):
    src3 = adj_list[t, 0].reshape(NS, NCH, CH)
    dst3 = adj_list[t, 1].reshape(NS, NCH, CH)
    gs = jnp.stack([x_list[t][:, :DH], x_list[t][:, DH:]], axis=0)
    deg = None
    acc = None
    for l in range(L - 1):
      if l == 0:
        acc, deg = prop_deg(src3, dst3, gs, ones_deg, zeros_deg)
      else:
        (acc,) = prop(src3, dst3, gs)
      gs = combine_split(acc, deg, W_gcn[l], b_gcn2[l])
    if L == 1:
      acc, deg = prop_deg(src3, dst3, gs, ones_deg, zeros_deg)
    else:
      (acc,) = prop(src3, dst3, gs)
    h = gru(acc, deg, W_gcn[L - 1], b_gcn2[L - 1], h, Wx, bx2, Wh, bh2)
    outs.append(h[:N])
  return jnp.stack(outs, axis=0)


# speedup vs baseline: 1.1671x; 1.1671x over previous
import functools

import jax
import jax.numpy as jnp
from jax import lax
from jax.experimental import pallas as pl
from jax.experimental.pallas import tpu as pltpu
from jax.experimental.pallas import tpu_sc as plsc

NC = 2
NS = 16
NW = NC * NS
CH = 250
DEGW = 8


ZCH = 128


NBUF = 2


def _emit_prop_job(c, s, refs, hbm, NCH, RPT, do_deg):
  (src_v, dst_v, rows_v, zrow_v, ones_v, acc_sh, deg_sh,
   gsems, ssems, dsem) = refs
  (src_hbm, dst_hbm, g_hbm, ones_hbm, zdeg_hbm, out_hbm, deg_hbm) = hbm
  NITER = NCH // NBUF
  ZREP = RPT // ZCH
  row0 = s * RPT

  for k in range(ZREP):
    pltpu.async_copy(zrow_v, acc_sh.at[pl.ds(row0 + k * ZCH, ZCH)], dsem)
    if do_deg:
      pltpu.async_copy(zdeg_hbm, deg_sh.at[pl.ds(row0 + k * ZCH, ZCH)], dsem)

  pltpu.sync_copy(src_hbm.at[s], src_v)
  pltpu.sync_copy(dst_hbm.at[s], dst_v)

  def start_gather(k, b):
    pltpu.async_copy(g_hbm.at[c].at[src_v.at[k]], rows_v.at[b], gsems[b])

  for b in range(NBUF):
    start_gather(b, b)
  if do_deg:
    pltpu.sync_copy(ones_hbm, ones_v)

  for k in range(ZREP):
    pltpu.make_async_copy(zrow_v, acc_sh.at[pl.ds(row0 + k * ZCH, ZCH)],
                          dsem).wait()
    if do_deg:
      pltpu.make_async_copy(zdeg_hbm, deg_sh.at[pl.ds(row0 + k * ZCH, ZCH)],
                            dsem).wait()
  plsc.subcore_barrier()

  def pipe_body(j, carry):
    base = j * NBUF
    for b in range(NBUF):
      k = base + b
      pltpu.make_async_copy(g_hbm.at[c].at[src_v.at[k]], rows_v.at[b],
                            gsems[b]).wait()
      pltpu.async_copy(rows_v.at[b], acc_sh.at[dst_v.at[k]], ssems[b],
                       add=True)
      if do_deg:
        @pl.when((c == 0) == (k < NCH // 2))
        def _():
          pltpu.async_copy(ones_v, deg_sh.at[dst_v.at[k]], dsem,
                           add=True).wait()
      pltpu.make_async_copy(rows_v.at[b], acc_sh.at[dst_v.at[k]],
                            ssems[b]).wait()

      @pl.when(j < NITER - 1)
      def _():
        start_gather(k + NBUF, b)
    return carry

  lax.fori_loop(0, NITER, pipe_body, 0)
  plsc.subcore_barrier()

  pltpu.sync_copy(acc_sh.at[pl.ds(row0, RPT)],
                  out_hbm.at[c, pl.ds(row0, RPT), :])
  if do_deg:
    pltpu.sync_copy(deg_sh.at[pl.ds(row0, RPT)],
                    deg_hbm.at[c, pl.ds(row0, RPT), :])


def _prop_scratch(NCH, DH, NP):
  return [
      pltpu.VMEM((NCH, CH), jnp.int32),
      pltpu.VMEM((NCH, CH), jnp.int32),
      pltpu.VMEM((NBUF, CH, DH), jnp.float32),
      pltpu.VMEM((ZCH, DH), jnp.float32),
      pltpu.VMEM((CH, DEGW), jnp.float32),
      pltpu.VMEM_SHARED((NP, DH), jnp.float32),
      pltpu.VMEM_SHARED((NP, DEGW), jnp.float32),
  ] + [pltpu.SemaphoreType.DMA] * (2 * NBUF + 1)


def _fill_zrow(zrow_v, DH):
  z16 = jnp.zeros((16,), jnp.float32)

  def fill_z(i, carry):
    for j in range(DH // 16):
      zrow_v[i, pl.ds(j * 16, 16)] = z16
    return carry

  lax.fori_loop(0, ZCH, fill_z, 0)


@functools.lru_cache(maxsize=None)
def _make_prop(NT, NP, D, E, with_deg):
  DH = D // 2
  assert E % (NS * CH) == 0
  NCH = E // (NS * CH)
  assert NCH % NBUF == 0
  assert NP % (NS * ZCH) == 0
  RPT = NP // NS

  mesh = plsc.VectorSubcoreMesh(core_axis_name="c", subcore_axis_name="s")
  out_type = [jax.ShapeDtypeStruct((NC, NP, DH), jnp.float32)]
  if with_deg:
    out_type.append(jax.ShapeDtypeStruct((NC, NP, DEGW), jnp.float32))

  def body(src_hbm, dst_hbm, g_hbm, *rest):
    if with_deg:
      ones_hbm, zdeg_hbm = rest[0], rest[1]
      out_hbm, deg_hbm = rest[2], rest[3]
      scr = rest[4:]
    else:
      ones_hbm = zdeg_hbm = deg_hbm = None
      out_hbm = rest[0]
      scr = rest[1:]
    refs = tuple(scr[:7]) + (scr[7:7 + NBUF], scr[7 + NBUF:7 + 2 * NBUF],
                             scr[7 + 2 * NBUF])
    c = lax.axis_index("c")
    s = lax.axis_index("s")
    _fill_zrow(refs[3], DH)
    _emit_prop_job(c, s, refs,
                   (src_hbm, dst_hbm, g_hbm, ones_hbm, zdeg_hbm, out_hbm,
                    deg_hbm), NCH, RPT, with_deg)

  return pl.kernel(body, out_type=out_type, mesh=mesh,
                   scratch_types=_prop_scratch(NCH, DH, NP),
                   compiler_params=pltpu.CompilerParams(
                       use_tc_tiling_on_sc=False),
                   name="sc_prop_deg" if with_deg else "sc_prop")


@functools.lru_cache(maxsize=None)
def _make_prop_pair(NT_A, NT_B, NP, D, E):
  DH = D // 2
  assert E % (NS * CH) == 0
  NCH = E // (NS * CH)
  assert NCH % NBUF == 0
  assert NP % (NS * ZCH) == 0
  RPT = NP // NS

  mesh = plsc.VectorSubcoreMesh(core_axis_name="c", subcore_axis_name="s")
  out_type = [
      jax.ShapeDtypeStruct((NC, NP, DH), jnp.float32),
      jax.ShapeDtypeStruct((NC, NP, DH), jnp.float32),
      jax.ShapeDtypeStruct((NC, NP, DEGW), jnp.float32),
  ]

  def body(srcA, dstA, gA, srcB, dstB, gB, ones_hbm, zdeg_hbm,
           outA, outB, degB, *scr):
    refs = tuple(scr[:7]) + (scr[7:7 + NBUF], scr[7 + NBUF:7 + 2 * NBUF],
                             scr[7 + 2 * NBUF])
    c = lax.axis_index("c")
    s = lax.axis_index("s")
    _fill_zrow(refs[3], DH)
    _emit_prop_job(c, s, refs, (srcA, dstA, gA, None, None, outA, None),
                   NCH, RPT, False)
    _emit_prop_job(c, s, refs, (srcB, dstB, gB, ones_hbm, zdeg_hbm, outB,
                                degB), NCH, RPT, True)

  return pl.kernel(body, out_type=out_type, mesh=mesh,
                   scratch_types=_prop_scratch(NCH, DH, NP),
                   compiler_params=pltpu.CompilerParams(
                       use_tc_tiling_on_sc=False),
                   name="sc_prop_pair")


@functools.lru_cache(maxsize=None)
def _make_combine(N, D, OUT, BN, split_out):
  assert N % BN == 0

  DH = D // 2
  OH = OUT // 2

  def body(acc_ref, deg_ref, w_ref, b_ref, o_ref):
    deg = deg_ref[0, :, 0:1] + deg_ref[1, :, 0:1]
    scale = 1.0 / jnp.maximum(deg, 1.0)
    a0 = acc_ref[0] * scale
    a1 = acc_ref[1] * scale
    g = jnp.maximum(
        jnp.dot(a0, w_ref[:DH, :], preferred_element_type=jnp.float32)
        + jnp.dot(a1, w_ref[DH:, :], preferred_element_type=jnp.float32)
        + b_ref[...], 0.0)
    if split_out:
      o_ref[0] = g[:, :OH]
      o_ref[1] = g[:, OH:]
    else:
      o_ref[...] = g

  if split_out:
    out_spec = pl.BlockSpec((NC, BN, OH), lambda i: (0, i, 0))
    out_shape = jax.ShapeDtypeStruct((NC, N, OH), jnp.float32)
  else:
    out_spec = pl.BlockSpec((BN, OUT), lambda i: (i, 0))
    out_shape = jax.ShapeDtypeStruct((N, OUT), jnp.float32)

  return pl.pallas_call(
      body,
      grid=(N // BN,),
      in_specs=[
          pl.BlockSpec((NC, BN, DH), lambda i: (0, i, 0)),
          pl.BlockSpec((NC, BN, DEGW), lambda i: (0, i, 0)),
          pl.BlockSpec((D, OUT), lambda i: (0, 0)),
          pl.BlockSpec((1, OUT), lambda i: (0, 0)),
      ],
      out_specs=out_spec,
      out_shape=out_shape,
      name="tc_combine_split" if split_out else "tc_combine",
  )


@functools.lru_cache(maxsize=None)
def _make_gru(N, D, OUT, BN):
  assert N % BN == 0
  DH = D // 2

  def body(acc_ref, deg_ref, w_ref, b_ref, h_ref, wx_ref, bx_ref, wh_ref,
           bh_ref, o_ref):
    deg = deg_ref[0, :, 0:1] + deg_ref[1, :, 0:1]
    scale = 1.0 / jnp.maximum(deg, 1.0)
    a0 = acc_ref[0] * scale
    a1 = acc_ref[1] * scale
    g = jnp.maximum(
        jnp.dot(a0, w_ref[:DH, :], preferred_element_type=jnp.float32)
        + jnp.dot(a1, w_ref[DH:, :], preferred_element_type=jnp.float32)
        + b_ref[...], 0.0)
    h = h_ref[...]
    gate_x = jnp.dot(g, wx_ref[...],
                     preferred_element_type=jnp.float32) + bx_ref[...]
    gate_h = jnp.dot(h, wh_ref[...],
                     preferred_element_type=jnp.float32) + bh_ref[...]
    i_r = gate_x[:, :OUT]
    i_u = gate_x[:, OUT:2 * OUT]
    i_n = gate_x[:, 2 * OUT:]
    h_r = gate_h[:, :OUT]
    h_u = gate_h[:, OUT:2 * OUT]
    h_n = gate_h[:, 2 * OUT:]
    r = jax.nn.sigmoid(i_r + h_r)
    u = jax.nn.sigmoid(i_u + h_u)
    cand = jnp.tanh(i_n + r * h_n)
    o_ref[...] = cand + u * (h - cand)

  return pl.pallas_call(
      body,
      grid=(N // BN,),
      in_specs=[
          pl.BlockSpec((NC, BN, DH), lambda i: (0, i, 0)),
          pl.BlockSpec((NC, BN, DEGW), lambda i: (0, i, 0)),
          pl.BlockSpec((D, OUT), lambda i: (0, 0)),
          pl.BlockSpec((1, OUT), lambda i: (0, 0)),
          pl.BlockSpec((BN, OUT), lambda i: (i, 0)),
          pl.BlockSpec((OUT, 3 * OUT), lambda i: (0, 0)),
          pl.BlockSpec((1, 3 * OUT), lambda i: (0, 0)),
          pl.BlockSpec((OUT, 3 * OUT), lambda i: (0, 0)),
          pl.BlockSpec((1, 3 * OUT), lambda i: (0, 0)),
      ],
      out_specs=pl.BlockSpec((BN, OUT), lambda i: (i, 0)),
      out_shape=jax.ShapeDtypeStruct((N, OUT), jnp.float32),
      name="tc_gru",
  )


def kernel(x_list, adj_list, W_gcn, b_gcn, Wx, bx, Wh, bh):
  T, N, D = x_list.shape
  L, _, OUT = W_gcn.shape
  E = adj_list.shape[2]
  ALIGN = NS * ZCH
  NP = ((N + ALIGN - 1) // ALIGN) * ALIGN
  BN = 1024
  assert NP % BN == 0
  NCH = E // (NS * CH)
  DH = D // 2

  prop_deg = _make_prop(N, NP, D, E, True)
  prop = _make_prop(NP, NP, OUT, E, False)
  prop_pair = _make_prop_pair(NP, N, NP, D, E)
  combine_split = _make_combine(NP, D, OUT, BN, True)
  gru = _make_gru(NP, D, OUT, BN)

  b_gcn2 = b_gcn.reshape(L, 1, OUT)
  bx2 = bx.reshape(1, 3 * OUT)
  bh2 = bh.reshape(1, 3 * OUT)

  ones_deg = jnp.ones((CH, DEGW), jnp.float32)
  zeros_deg = jnp.zeros((ZCH, DEGW), jnp.float32)
  h = jnp.zeros((NP, OUT), x_list.dtype)
  outs = []

  srcs = [adj_list[t, 0].reshape(NS, NCH, CH) for t in range(T)]
  dsts = [adj_list[t, 1].reshape(NS, NCH, CH) for t in range(T)]
  xsp = [jnp.stack([x_list[t][:, :DH], x_list[t][:, DH:]], axis=0)
         for t in range(T)]

  if L == 2:
    acc, deg = prop_deg(srcs[0], dsts[0], xsp[0], ones_deg, zeros_deg)
    gs = combine_split(acc, deg, W_gcn[0], b_gcn2[0])
    for t in range(T):
      if t < T - 1:
        accA, accB, deg_next = prop_pair(srcs[t], dsts[t], gs,
                                         srcs[t + 1], dsts[t + 1],
                                         xsp[t + 1], ones_deg, zeros_deg)
      else:
        (accA,) = prop(srcs[t], dsts[t], gs)
        accB = deg_next = None
      h = gru(accA, deg, W_gcn[1], b_gcn2[1], h, Wx, bx2, Wh, bh2)
      outs.append(h[:N])
      if t < T - 1:
        gs = combine_split(accB, deg_next, W_gcn[0], b_gcn2[0])
        deg = deg_next
  else:
    for t in range(T):
      gs = xsp[t]
      deg = None
      acc = None
      for l in range(L - 1):
        if l == 0:
          acc, deg = prop_deg(srcs[t], dsts[t], gs, ones_deg, zeros_deg)
        else:
          (acc,) = prop(srcs[t], dsts[t], gs)
        gs = combine_split(acc, deg, W_gcn[l], b_gcn2[l])
      if L == 1:
        acc, deg = prop_deg(srcs[t], dsts[t], gs, ones_deg, zeros_deg)
      else:
        (acc,) = prop(srcs[t], dsts[t], gs)
      h = gru(acc, deg, W_gcn[L - 1], b_gcn2[L - 1], h, Wx, bx2, Wh, bh2)
      outs.append(h[:N])
  return jnp.stack(outs, axis=0)


# speedup vs baseline: 1.2506x; 1.0715x over previous
import functools

import jax
import jax.numpy as jnp
from jax import lax
from jax.experimental import pallas as pl
from jax.experimental.pallas import tpu as pltpu
from jax.experimental.pallas import tpu_sc as plsc

NC = 2
NS = 16
NW = NC * NS
CH = 250
DEGW = 8


ZCH = 128


NBUF = 2


def _emit_prop_job(c, s, refs, hbm, NCH, RPT, do_deg):
  (src_v, dst_v, rows_v, zrow_v, ones_v, acc_sh, deg_sh,
   gsems, ssems, dsem) = refs
  (src_hbm, dst_hbm, g_hbm, ones_hbm, zdeg_hbm, out_hbm, deg_hbm) = hbm
  NITER = NCH // NBUF
  ZREP = RPT // ZCH
  row0 = s * RPT

  for k in range(ZREP):
    pltpu.async_copy(zrow_v, acc_sh.at[pl.ds(row0 + k * ZCH, ZCH)], dsem)
    if do_deg:
      pltpu.async_copy(zdeg_hbm, deg_sh.at[pl.ds(row0 + k * ZCH, ZCH)], dsem)

  pltpu.sync_copy(src_hbm.at[s], src_v)
  pltpu.sync_copy(dst_hbm.at[s], dst_v)

  def start_gather(k, b):
    pltpu.async_copy(g_hbm.at[c].at[src_v.at[k]], rows_v.at[b], gsems[b])

  for b in range(NBUF):
    start_gather(b, b)
  if do_deg:
    pltpu.sync_copy(ones_hbm, ones_v)

  for k in range(ZREP):
    pltpu.make_async_copy(zrow_v, acc_sh.at[pl.ds(row0 + k * ZCH, ZCH)],
                          dsem).wait()
    if do_deg:
      pltpu.make_async_copy(zdeg_hbm, deg_sh.at[pl.ds(row0 + k * ZCH, ZCH)],
                            dsem).wait()
  plsc.subcore_barrier()

  def pipe_body(j, carry):
    base = j * NBUF
    for b in range(NBUF):
      k = base + b
      pltpu.make_async_copy(g_hbm.at[c].at[src_v.at[k]], rows_v.at[b],
                            gsems[b]).wait()
      pltpu.async_copy(rows_v.at[b], acc_sh.at[dst_v.at[k]], ssems[b],
                       add=True)
      if do_deg:
        @pl.when((c == 0) == (k < NCH // 2))
        def _():
          pltpu.async_copy(ones_v, deg_sh.at[dst_v.at[k]], dsem,
                           add=True).wait()
      pltpu.make_async_copy(rows_v.at[b], acc_sh.at[dst_v.at[k]],
                            ssems[b]).wait()

      @pl.when(j < NITER - 1)
      def _():
        start_gather(k + NBUF, b)
    return carry

  lax.fori_loop(0, NITER, pipe_body, 0)
  plsc.subcore_barrier()

  pltpu.sync_copy(acc_sh.at[pl.ds(row0, RPT)],
                  out_hbm.at[c, pl.ds(row0, RPT), :])
  if do_deg:
    pltpu.sync_copy(deg_sh.at[pl.ds(row0, RPT)],
                    deg_hbm.at[c, pl.ds(row0, RPT), :])


def _prop_scratch(NCH, DH, NP):
  return [
      pltpu.VMEM((NCH, CH), jnp.int32),
      pltpu.VMEM((NCH, CH), jnp.int32),
      pltpu.VMEM((NBUF, CH, DH), jnp.float32),
      pltpu.VMEM((ZCH, DH), jnp.float32),
      pltpu.VMEM((CH, DEGW), jnp.float32),
      pltpu.VMEM_SHARED((NP, DH), jnp.float32),
      pltpu.VMEM_SHARED((NP, DEGW), jnp.float32),
  ] + [pltpu.SemaphoreType.DMA] * (2 * NBUF + 1)


def _fill_zrow(zrow_v, DH):
  z16 = jnp.zeros((16,), jnp.float32)

  def fill_z(i, carry):
    for j in range(DH // 16):
      zrow_v[i, pl.ds(j * 16, 16)] = z16
    return carry

  lax.fori_loop(0, ZCH, fill_z, 0)


@functools.lru_cache(maxsize=None)
def _make_prop(NT, NP, D, E, with_deg):
  DH = D // 2
  assert E % (NS * CH) == 0
  NCH = E // (NS * CH)
  assert NCH % NBUF == 0
  assert NP % (NS * ZCH) == 0
  RPT = NP // NS

  mesh = plsc.VectorSubcoreMesh(core_axis_name="c", subcore_axis_name="s")
  out_type = [jax.ShapeDtypeStruct((NC, NP, DH), jnp.float32)]
  if with_deg:
    out_type.append(jax.ShapeDtypeStruct((NC, NP, DEGW), jnp.float32))

  def body(src_hbm, dst_hbm, g_hbm, *rest):
    if with_deg:
      ones_hbm, zdeg_hbm = rest[0], rest[1]
      out_hbm, deg_hbm = rest[2], rest[3]
      scr = rest[4:]
    else:
      ones_hbm = zdeg_hbm = deg_hbm = None
      out_hbm = rest[0]
      scr = rest[1:]
    refs = tuple(scr[:7]) + (scr[7:7 + NBUF], scr[7 + NBUF:7 + 2 * NBUF],
                             scr[7 + 2 * NBUF])
    c = lax.axis_index("c")
    s = lax.axis_index("s")
    _fill_zrow(refs[3], DH)
    _emit_prop_job(c, s, refs,
                   (src_hbm, dst_hbm, g_hbm, ones_hbm, zdeg_hbm, out_hbm,
                    deg_hbm), NCH, RPT, with_deg)

  return pl.kernel(body, out_type=out_type, mesh=mesh,
                   scratch_types=_prop_scratch(NCH, DH, NP),
                   compiler_params=pltpu.CompilerParams(
                       use_tc_tiling_on_sc=False),
                   name="sc_prop_deg" if with_deg else "sc_prop")


@functools.lru_cache(maxsize=None)
def _make_prop_pair(NT_A, NT_B, NP, D, E):
  DH = D // 2
  assert E % (NS * CH) == 0
  NCH = E // (NS * CH)
  assert NCH % NBUF == 0
  assert NP % (NS * ZCH) == 0
  RPT = NP // NS

  mesh = plsc.VectorSubcoreMesh(core_axis_name="c", subcore_axis_name="s")
  out_type = [
      jax.ShapeDtypeStruct((NC, NP, DH), jnp.float32),
      jax.ShapeDtypeStruct((NC, NP, DH), jnp.float32),
      jax.ShapeDtypeStruct((NC, NP, DEGW), jnp.float32),
  ]

  def body(srcA, dstA, gA, srcB, dstB, gB, ones_hbm, zdeg_hbm,
           outA, outB, degB, *scr):
    refs = tuple(scr[:7]) + (scr[7:7 + NBUF], scr[7 + NBUF:7 + 2 * NBUF],
                             scr[7 + 2 * NBUF])
    c = lax.axis_index("c")
    s = lax.axis_index("s")
    _fill_zrow(refs[3], DH)
    _emit_prop_job(c, s, refs, (srcA, dstA, gA, None, None, outA, None),
                   NCH, RPT, False)
    _emit_prop_job(c, s, refs, (srcB, dstB, gB, ones_hbm, zdeg_hbm, outB,
                                degB), NCH, RPT, True)

  return pl.kernel(body, out_type=out_type, mesh=mesh,
                   scratch_types=_prop_scratch(NCH, DH, NP),
                   compiler_params=pltpu.CompilerParams(
                       use_tc_tiling_on_sc=False),
                   name="sc_prop_pair")


@functools.lru_cache(maxsize=None)
def _make_combine(N, D, OUT, BN, split_out):
  assert N % BN == 0

  DH = D // 2
  OH = OUT // 2

  def body(acc_ref, deg_ref, w_ref, b_ref, o_ref):
    deg = deg_ref[0, :, 0:1] + deg_ref[1, :, 0:1]
    scale = 1.0 / jnp.maximum(deg, 1.0)
    a0 = acc_ref[0] * scale
    a1 = acc_ref[1] * scale
    g = jnp.maximum(
        jnp.dot(a0, w_ref[:DH, :], preferred_element_type=jnp.float32)
        + jnp.dot(a1, w_ref[DH:, :], preferred_element_type=jnp.float32)
        + b_ref[...], 0.0)
    if split_out:
      o_ref[0] = g[:, :OH]
      o_ref[1] = g[:, OH:]
    else:
      o_ref[...] = g

  if split_out:
    out_spec = pl.BlockSpec((NC, BN, OH), lambda i: (0, i, 0))
    out_shape = jax.ShapeDtypeStruct((NC, N, OH), jnp.float32)
  else:
    out_spec = pl.BlockSpec((BN, OUT), lambda i: (i, 0))
    out_shape = jax.ShapeDtypeStruct((N, OUT), jnp.float32)

  return pl.pallas_call(
      body,
      grid=(N // BN,),
      in_specs=[
          pl.BlockSpec((NC, BN, DH), lambda i: (0, i, 0)),
          pl.BlockSpec((NC, BN, DEGW), lambda i: (0, i, 0)),
          pl.BlockSpec((D, OUT), lambda i: (0, 0)),
          pl.BlockSpec((1, OUT), lambda i: (0, 0)),
      ],
      out_specs=out_spec,
      out_shape=out_shape,
      name="tc_combine_split" if split_out else "tc_combine",
  )


@functools.lru_cache(maxsize=None)
def _make_gru(N, D, OUT, BN):
  assert N % BN == 0
  DH = D // 2

  def body(acc_ref, deg_ref, w_ref, b_ref, h_ref, wx_ref, bx_ref, wh_ref,
           bh_ref, o_ref):
    deg = deg_ref[0, :, 0:1] + deg_ref[1, :, 0:1]
    scale = 1.0 / jnp.maximum(deg, 1.0)
    a0 = acc_ref[0] * scale
    a1 = acc_ref[1] * scale
    g = jnp.maximum(
        jnp.dot(a0, w_ref[:DH, :], preferred_element_type=jnp.float32)
        + jnp.dot(a1, w_ref[DH:, :], preferred_element_type=jnp.float32)
        + b_ref[...], 0.0)
    h = h_ref[...]
    gate_x = jnp.dot(g, wx_ref[...],
                     preferred_element_type=jnp.float32) + bx_ref[...]
    gate_h = jnp.dot(h, wh_ref[...],
                     preferred_element_type=jnp.float32) + bh_ref[...]
    i_r = gate_x[:, :OUT]
    i_u = gate_x[:, OUT:2 * OUT]
    i_n = gate_x[:, 2 * OUT:]
    h_r = gate_h[:, :OUT]
    h_u = gate_h[:, OUT:2 * OUT]
    h_n = gate_h[:, 2 * OUT:]
    r = jax.nn.sigmoid(i_r + h_r)
    u = jax.nn.sigmoid(i_u + h_u)
    cand = jnp.tanh(i_n + r * h_n)
    o_ref[...] = cand + u * (h - cand)

  return pl.pallas_call(
      body,
      grid=(N // BN,),
      in_specs=[
          pl.BlockSpec((NC, BN, DH), lambda i: (0, i, 0)),
          pl.BlockSpec((NC, BN, DEGW), lambda i: (0, i, 0)),
          pl.BlockSpec((D, OUT), lambda i: (0, 0)),
          pl.BlockSpec((1, OUT), lambda i: (0, 0)),
          pl.BlockSpec((BN, OUT), lambda i: (i, 0)),
          pl.BlockSpec((OUT, 3 * OUT), lambda i: (0, 0)),
          pl.BlockSpec((1, 3 * OUT), lambda i: (0, 0)),
          pl.BlockSpec((OUT, 3 * OUT), lambda i: (0, 0)),
          pl.BlockSpec((1, 3 * OUT), lambda i: (0, 0)),
      ],
      out_specs=pl.BlockSpec((BN, OUT), lambda i: (i, 0)),
      out_shape=jax.ShapeDtypeStruct((N, OUT), jnp.float32),
      name="tc_gru",
  )


def kernel(x_list, adj_list, W_gcn, b_gcn, Wx, bx, Wh, bh):
  T, N, D = x_list.shape
  L, _, OUT = W_gcn.shape
  E = adj_list.shape[2]
  ALIGN = NS * ZCH
  NP = ((N + ALIGN - 1) // ALIGN) * ALIGN
  BN = 1024
  assert NP % BN == 0
  NCH = E // (NS * CH)
  DH = D // 2

  prop_deg = _make_prop(N, NP, D, E, True)
  prop = _make_prop(NP, NP, OUT, E, False)
  prop_pair = _make_prop_pair(NP, N, NP, D, E)
  combine_split = _make_combine(NP, D, OUT, BN, True)
  gru = _make_gru(NP, D, OUT, BN)

  b_gcn2 = b_gcn.reshape(L, 1, OUT)
  bx2 = bx.reshape(1, 3 * OUT)
  bh2 = bh.reshape(1, 3 * OUT)

  ones_deg = jnp.ones((CH, DEGW), jnp.float32)
  zeros_deg = jnp.zeros((ZCH, DEGW), jnp.float32)
  h = jnp.zeros((NP, OUT), x_list.dtype)
  outs = []

  srcs = [adj_list[t, 0].reshape(NS, NCH, CH) for t in range(T)]
  dsts = [adj_list[t, 1].reshape(NS, NCH, CH) for t in range(T)]
  xsp = [jnp.stack([x_list[t][:, :DH], x_list[t][:, DH:]], axis=0)
         for t in range(T)]

  if False and L == 2:
    acc, deg = prop_deg(srcs[0], dsts[0], xsp[0], ones_deg, zeros_deg)
    gs = combine_split(acc, deg, W_gcn[0], b_gcn2[0])
    for t in range(T):
      if t < T - 1:
        accA, accB, deg_next = prop_pair(srcs[t], dsts[t], gs,
                                         srcs[t + 1], dsts[t + 1],
                                         xsp[t + 1], ones_deg, zeros_deg)
      else:
        (accA,) = prop(srcs[t], dsts[t], gs)
        accB = deg_next = None
      h = gru(accA, deg, W_gcn[1], b_gcn2[1], h, Wx, bx2, Wh, bh2)
      outs.append(h[:N])
      if t < T - 1:
        gs = combine_split(accB, deg_next, W_gcn[0], b_gcn2[0])
        deg = deg_next
  else:
    for t in range(T):
      gs = xsp[t]
      deg = None
      acc = None
      for l in range(L - 1):
        if l == 0:
          acc, deg = prop_deg(srcs[t], dsts[t], gs, ones_deg, zeros_deg)
        else:
          (acc,) = prop(srcs[t], dsts[t], gs)
        gs = combine_split(acc, deg, W_gcn[l], b_gcn2[l])
      if L == 1:
        acc, deg = prop_deg(srcs[t], dsts[t], gs, ones_deg, zeros_deg)
      else:
        (acc,) = prop(srcs[t], dsts[t], gs)
      h = gru(acc, deg, W_gcn[L - 1], b_gcn2[L - 1], h, Wx, bx2, Wh, bh2)
      outs.append(h[:N])
  return jnp.stack(outs, axis=0)
